# baseline (speedup 1.0000x reference)
import jax
import jax.numpy as jnp
from jax import lax
from jax.experimental import pallas as pl
from jax.experimental.pallas import tpu as pltpu
from jax.experimental.pallas import tpu_sc as plsc

NC, NS, L = 2, 16, 16
NW = NC * NS
LEVELS = (4096, 2048, 1024, 512)
CH = 256
SPL = 128
NT = 16


def _tap_math(u, v, S):
    Sf = jnp.float32(S)
    ix = ((u * 2.0 - 1.0 + 1.0) * Sf - 1.0) * 0.5
    iy = ((v * 2.0 - 1.0 + 1.0) * Sf - 1.0) * 0.5
    ix = jnp.minimum(jnp.maximum(ix, -1.0), Sf)
    iy = jnp.minimum(jnp.maximum(iy, -1.0), Sf)
    x0 = (ix + 1.0).astype(jnp.int32) - 1
    y0 = (iy + 1.0).astype(jnp.int32) - 1
    fx1 = ix - x0.astype(jnp.float32)
    fx0 = 1.0 - fx1
    fy1 = iy - y0.astype(jnp.float32)
    fy0 = 1.0 - fy1
    x1 = x0 + 1
    y1 = y0 + 1
    zero = jnp.zeros_like(fx1)
    wx0 = jnp.where((x0 >= 0) & (x0 <= S - 1), fx0, zero)
    wx1 = jnp.where(x1 <= S - 1, fx1, zero)
    wy0 = jnp.where((y0 >= 0) & (y0 <= S - 1), fy0, zero)
    wy1 = jnp.where(y1 <= S - 1, fy1, zero)
    x0c = jnp.minimum(jnp.maximum(x0, 0), S - 1)
    x1c = jnp.minimum(jnp.maximum(x1, 0), S - 1)
    y0c = jnp.minimum(jnp.maximum(y0, 0), S - 1)
    y1c = jnp.minimum(jnp.maximum(y1, 0), S - 1)
    yb0 = y0c * S
    yb1 = y1c * S
    idx = (yb0 + x0c, yb0 + x1c, yb1 + x0c, yb1 + x1c)
    w = (wx0 * wy0, wx1 * wy0, wx0 * wy1, wx1 * wy1)
    return idx, w


def _tex_body(uv_hbm, t0, t1, t2, t3, out_hbm,
              uvs, idxs, wts, vals, outv, sem0, sem1, osem0, osem1):
    wid = lax.axis_index("s") * NC + lax.axis_index("c")
    npts = uv_hbm.shape[0] // 2
    per_w = npts // NW
    nch = per_w // CH
    wbase = wid * per_w
    texs = (t0, t1, t2, t3)
    sems = (sem0, sem1)
    osems = (osem0, osem1)

    pltpu.sync_copy(uv_hbm.at[pl.ds(wbase * 2, per_w * 2)], uvs)

    def gather_copies(p):
        cps = []
        for l in range(4):
            for t in range(4):
                T = l * 4 + t
                for s in range(CH // SPL):
                    cps.append(pltpu.make_async_copy(
                        texs[l].at[idxs.at[p, T, pl.ds(s * SPL, SPL)]],
                        vals.at[p, T, pl.ds(s * SPL, SPL)],
                        sems[p]))
        return cps

    def prep(c, p):
        coff = c * CH

        def group_body(g, carry):
            off = coff + g * L
            lane2 = off * 2 + jnp.arange(0, 2 * L, 2, dtype=jnp.int32)
            u = plsc.load_gather(uvs, [lane2])
            v = plsc.load_gather(uvs, [lane2 + 1])
            soff = g * L
            for l, S in enumerate(LEVELS):
                idx, w = _tap_math(u, v, S)
                for t in range(4):
                    T = l * 4 + t
                    idxs[p, T, pl.ds(soff, L)] = idx[t]
                    wts[p, T, pl.ds(soff, L)] = w[t]
            return carry

        lax.fori_loop(0, CH // L, group_body, 0)
        for cp in gather_copies(p):
            cp.start()

    def finish(c, p):
        for cp in gather_copies(p):
            cp.wait()

        @pl.when(c >= 2)
        def _():
            pltpu.make_async_copy(
                outv.at[p], out_hbm.at[pl.ds(wbase + (c - 2) * CH, CH)],
                osems[p]).wait()

        def acc_body(g, carry):
            off = g * L
            a = wts[p, 0, pl.ds(off, L)] * vals[p, 0, pl.ds(off, L)]
            for T in range(1, NT):
                a = a + wts[p, T, pl.ds(off, L)] * vals[p, T, pl.ds(off, L)]
            outv[p, pl.ds(off, L)] = a
            return carry

        lax.fori_loop(0, CH // L, acc_body, 0)
        pltpu.make_async_copy(
            outv.at[p], out_hbm.at[pl.ds(wbase + c * CH, CH)],
            osems[p]).start()

    npair = nch // 2
    prep(0, 0)

    def pair_body(i, carry):
        c0 = 2 * i
        prep(c0 + 1, 1)
        finish(c0, 0)

        @pl.when(i < npair - 1)
        def _():
            prep(c0 + 2, 0)

        finish(c0 + 1, 1)
        return carry

    lax.fori_loop(0, npair, pair_body, 0)

    pltpu.make_async_copy(
        outv.at[0], out_hbm.at[pl.ds(wbase + (nch - 2) * CH, CH)],
        osems[0]).wait()
    pltpu.make_async_copy(
        outv.at[1], out_hbm.at[pl.ds(wbase + (nch - 1) * CH, CH)],
        osems[1]).wait()


def kernel(x, layer1, layer2, layer3, layer4):
    B, H, W, _ = x.shape
    N = B * H * W
    per_w = N // NW
    uv = x.reshape(N * 2)
    texs = [t.reshape(-1) for t in (layer1, layer2, layer3, layer4)]
    mesh = plsc.VectorSubcoreMesh(core_axis_name="c", subcore_axis_name="s")
    run = pl.kernel(
        _tex_body,
        out_type=jax.ShapeDtypeStruct((N,), jnp.float32),
        mesh=mesh,
        compiler_params=pltpu.CompilerParams(needs_layout_passes=False),
        scratch_types=[
            pltpu.VMEM((per_w * 2,), jnp.float32),
            pltpu.VMEM((2, NT, CH), jnp.int32),
            pltpu.VMEM((2, NT, CH), jnp.float32),
            pltpu.VMEM((2, NT, CH), jnp.float32),
            pltpu.VMEM((2, CH), jnp.float32),
            pltpu.SemaphoreType.DMA,
            pltpu.SemaphoreType.DMA,
            pltpu.SemaphoreType.DMA,
            pltpu.SemaphoreType.DMA,
        ],
    )
    y = run(uv, *texs)
    return y.reshape(B, 1, H, W)


# speedup vs baseline: 2.8156x; 2.8156x over previous
import jax
import jax.numpy as jnp
from jax import lax
from jax.experimental import pallas as pl
from jax.experimental.pallas import tpu as pltpu
from jax.experimental.pallas import tpu_sc as plsc

NC, NS, L = 2, 16, 16
NW = NC * NS
LEVELS = (4096, 2048, 1024, 512)
SPL = 128
CH = 512
NT = 16


def _tap_math(u, v, S):
    Sf = jnp.float32(S)
    ix = ((u * 2.0 - 1.0 + 1.0) * Sf - 1.0) * 0.5
    iy = ((v * 2.0 - 1.0 + 1.0) * Sf - 1.0) * 0.5
    ix = jnp.minimum(jnp.maximum(ix, -1.0), Sf)
    iy = jnp.minimum(jnp.maximum(iy, -1.0), Sf)
    x0 = (ix + 1.0).astype(jnp.int32) - 1
    y0 = (iy + 1.0).astype(jnp.int32) - 1
    fx1 = ix - x0.astype(jnp.float32)
    fx0 = 1.0 - fx1
    fy1 = iy - y0.astype(jnp.float32)
    fy0 = 1.0 - fy1
    x1 = x0 + 1
    y1 = y0 + 1
    zero = jnp.zeros_like(fx1)
    wx0 = jnp.where((x0 >= 0) & (x0 <= S - 1), fx0, zero)
    wx1 = jnp.where(x1 <= S - 1, fx1, zero)
    wy0 = jnp.where((y0 >= 0) & (y0 <= S - 1), fy0, zero)
    wy1 = jnp.where(y1 <= S - 1, fy1, zero)
    x0c = jnp.minimum(jnp.maximum(x0, 0), S - 1)
    x1c = jnp.minimum(jnp.maximum(x1, 0), S - 1)
    y0c = jnp.minimum(jnp.maximum(y0, 0), S - 1)
    y1c = jnp.minimum(jnp.maximum(y1, 0), S - 1)
    yb0 = y0c * S
    yb1 = y1c * S
    idx = (yb0 + x0c, yb0 + x1c, yb1 + x0c, yb1 + x1c)
    w = (wx0 * wy0, wx1 * wy0, wx0 * wy1, wx1 * wy1)
    return idx, w


def _tex_body(u_hbm, v_hbm, t0, t1, t2, t3, out_hbm,
              uu, vv, idxs, wts, vals, outv,
              sem0, sem1, osem0, osem1):
    wid = lax.axis_index("s") * NC + lax.axis_index("c")
    npts = u_hbm.shape[0]
    per_w = npts // NW
    nch = per_w // CH
    wbase = wid * per_w
    texs = (t0, t1, t2, t3)
    sems = (sem0, sem1)
    osems = (osem0, osem1)

    pltpu.sync_copy(u_hbm.at[pl.ds(wbase, per_w)], uu)
    pltpu.sync_copy(v_hbm.at[pl.ds(wbase, per_w)], vv)

    def gather_copies(p):
        cps = []
        for l in range(4):
            for t in range(4):
                T = l * 4 + t
                for s in range(CH // SPL):
                    cps.append(pltpu.make_async_copy(
                        texs[l].at[idxs.at[p, T, pl.ds(s * SPL, SPL)]],
                        vals.at[p, T, pl.ds(s * SPL, SPL)],
                        sems[p]))
        return cps

    def prep(c, p):
        coff = c * CH

        def group_body(g, carry):
            soff = g * L
            off = coff + soff
            u = uu[pl.ds(off, L)]
            v = vv[pl.ds(off, L)]
            for l, S in enumerate(LEVELS):
                idx, w = _tap_math(u, v, S)
                for t in range(4):
                    T = l * 4 + t
                    idxs[p, T, pl.ds(soff, L)] = idx[t]
                    wts[p, T, pl.ds(soff, L)] = w[t]
            return carry

        lax.fori_loop(0, CH // L, group_body, 0)
        for cp in gather_copies(p):
            cp.start()

    def finish(c, p):
        for cp in gather_copies(p):
            cp.wait()

        @pl.when(c >= 2)
        def _():
            pltpu.make_async_copy(
                outv.at[p], out_hbm.at[pl.ds(wbase + (c - 2) * CH, CH)],
                osems[p]).wait()

        def group_body(g, carry):
            soff = g * L
            a = wts[p, 0, pl.ds(soff, L)] * vals[p, 0, pl.ds(soff, L)]
            for T in range(1, NT):
                a = a + wts[p, T, pl.ds(soff, L)] * vals[p, T, pl.ds(soff, L)]
            outv[p, pl.ds(soff, L)] = a
            return carry

        lax.fori_loop(0, CH // L, group_body, 0)
        pltpu.make_async_copy(
            outv.at[p], out_hbm.at[pl.ds(wbase + c * CH, CH)],
            osems[p]).start()

    npair = nch // 2
    prep(0, 0)

    def pair_body(i, carry):
        c0 = 2 * i
        prep(c0 + 1, 1)
        finish(c0, 0)

        @pl.when(i < npair - 1)
        def _():
            prep(c0 + 2, 0)

        finish(c0 + 1, 1)
        return carry

    lax.fori_loop(0, npair, pair_body, 0)

    pltpu.make_async_copy(
        outv.at[0], out_hbm.at[pl.ds(wbase + (nch - 2) * CH, CH)],
        osems[0]).wait()
    pltpu.make_async_copy(
        outv.at[1], out_hbm.at[pl.ds(wbase + (nch - 1) * CH, CH)],
        osems[1]).wait()


def kernel(x, layer1, layer2, layer3, layer4):
    B, H, W, _ = x.shape
    N = B * H * W
    per_w = N // NW
    uv = x.reshape(N, 2).T
    texs = [t.reshape(-1) for t in (layer1, layer2, layer3, layer4)]
    mesh = plsc.VectorSubcoreMesh(core_axis_name="c", subcore_axis_name="s")
    run = pl.kernel(
        _tex_body,
        out_type=jax.ShapeDtypeStruct((N,), jnp.float32),
        mesh=mesh,
        compiler_params=pltpu.CompilerParams(needs_layout_passes=False),
        scratch_types=[
            pltpu.VMEM((per_w,), jnp.float32),
            pltpu.VMEM((per_w,), jnp.float32),
            pltpu.VMEM((2, NT, CH), jnp.int32),
            pltpu.VMEM((2, NT, CH), jnp.float32),
            pltpu.VMEM((2, NT, CH), jnp.float32),
            pltpu.VMEM((2, CH), jnp.float32),
            pltpu.SemaphoreType.DMA,
            pltpu.SemaphoreType.DMA,
            pltpu.SemaphoreType.DMA,
            pltpu.SemaphoreType.DMA,
        ],
    )
    y = run(uv[0], uv[1], *texs)
    return y.reshape(B, 1, H, W)


# speedup vs baseline: 3.2817x; 1.1656x over previous
import jax
import jax.numpy as jnp
from jax import lax
from jax.experimental import pallas as pl
from jax.experimental.pallas import tpu as pltpu
from jax.experimental.pallas import tpu_sc as plsc

NC, NS, L = 2, 16, 16
NW = NC * NS
LEVELS = (4096, 2048, 1024, 512)
SPL = 128
CH = 512
NT = 16


def _tap_math(u, v, S):
    Sf = jnp.float32(S)
    ix = ((u * 2.0 - 1.0 + 1.0) * Sf - 1.0) * 0.5
    iy = ((v * 2.0 - 1.0 + 1.0) * Sf - 1.0) * 0.5
    ix = jnp.minimum(jnp.maximum(ix, -1.0), Sf)
    iy = jnp.minimum(jnp.maximum(iy, -1.0), Sf)
    x0 = (ix + 1.0).astype(jnp.int32) - 1
    y0 = (iy + 1.0).astype(jnp.int32) - 1
    fx1 = ix - x0.astype(jnp.float32)
    fx0 = 1.0 - fx1
    fy1 = iy - y0.astype(jnp.float32)
    fy0 = 1.0 - fy1
    x1 = x0 + 1
    y1 = y0 + 1
    zero = jnp.zeros_like(fx1)
    wx0 = jnp.where((x0 >= 0) & (x0 <= S - 1), fx0, zero)
    wx1 = jnp.where(x1 <= S - 1, fx1, zero)
    wy0 = jnp.where((y0 >= 0) & (y0 <= S - 1), fy0, zero)
    wy1 = jnp.where(y1 <= S - 1, fy1, zero)
    y0c = jnp.minimum(jnp.maximum(y0, 0), S - 1)
    y1c = jnp.minimum(jnp.maximum(y1, 0), S - 1)
    yb0 = y0c * S
    yb1 = y1c * S
    vmax = S * S - 1
    i00 = jnp.minimum(jnp.maximum(yb0 + x0, 0), vmax)
    i10 = jnp.minimum(jnp.maximum(yb1 + x0, 0), vmax)
    idx = (i00, i10)
    w = (wx0 * wy0, wx1 * wy0, wx0 * wy1, wx1 * wy1)
    return idx, w


def _tex_body(u_hbm, v_hbm, t0, t1, t2, t3, out_hbm,
              uu, vv, idxs, wts, vals, outv,
              sem0, sem1, osem0, osem1):
    wid = lax.axis_index("s") * NC + lax.axis_index("c")
    npts = u_hbm.shape[0]
    per_w = npts // NW
    nch = per_w // CH
    wbase = wid * per_w
    texs = (t0, t1, t2, t3)
    sems = (sem0, sem1)
    osems = (osem0, osem1)

    pltpu.sync_copy(u_hbm.at[pl.ds(wbase, per_w)], uu)
    pltpu.sync_copy(v_hbm.at[pl.ds(wbase, per_w)], vv)

    def gather_copies(p):
        cps = []
        for l in range(4):
            for t in range(2):
                T = l * 2 + t
                for s in range(CH // SPL):
                    cps.append(pltpu.make_async_copy(
                        texs[l].at[idxs.at[p, T, pl.ds(s * SPL, SPL)]],
                        vals.at[p, T, pl.ds(s * SPL, SPL)],
                        sems[p]))
        return cps

    def prep(c, p):
        coff = c * CH

        def group_body(g, carry):
            soff = g * L
            off = coff + soff
            u = uu[pl.ds(off, L)]
            v = vv[pl.ds(off, L)]
            for l, S in enumerate(LEVELS):
                idx, w = _tap_math(u, v, S)
                idxs[p, 2 * l, pl.ds(soff, L)] = idx[0]
                idxs[p, 2 * l + 1, pl.ds(soff, L)] = idx[1]
                for t in range(4):
                    wts[p, l * 4 + t, pl.ds(soff, L)] = w[t]
            return carry

        lax.fori_loop(0, CH // L, group_body, 0)
        for cp in gather_copies(p):
            cp.start()

    def finish(c, p):
        for cp in gather_copies(p):
            cp.wait()

        @pl.when(c >= 2)
        def _():
            pltpu.make_async_copy(
                outv.at[p], out_hbm.at[pl.ds(wbase + (c - 2) * CH, CH)],
                osems[p]).wait()

        himask = jnp.full((L,), -65536, jnp.int32)

        def group_body(g, carry):
            soff = g * L
            a = None
            for l in range(4):
                pk0 = vals[p, 2 * l, pl.ds(soff, L)]
                pk1 = vals[p, 2 * l + 1, pl.ds(soff, L)]
                f00 = lax.bitcast_convert_type(pk0 << 16, jnp.float32)
                f01 = lax.bitcast_convert_type(pk0 & himask, jnp.float32)
                f10 = lax.bitcast_convert_type(pk1 << 16, jnp.float32)
                f11 = lax.bitcast_convert_type(pk1 & himask, jnp.float32)
                part = (wts[p, l * 4 + 0, pl.ds(soff, L)] * f00
                        + wts[p, l * 4 + 1, pl.ds(soff, L)] * f01
                        + wts[p, l * 4 + 2, pl.ds(soff, L)] * f10
                        + wts[p, l * 4 + 3, pl.ds(soff, L)] * f11)
                a = part if a is None else a + part
            outv[p, pl.ds(soff, L)] = a
            return carry

        lax.fori_loop(0, CH // L, group_body, 0)
        pltpu.make_async_copy(
            outv.at[p], out_hbm.at[pl.ds(wbase + c * CH, CH)],
            osems[p]).start()

    npair = nch // 2
    prep(0, 0)

    def pair_body(i, carry):
        c0 = 2 * i
        prep(c0 + 1, 1)
        finish(c0, 0)

        @pl.when(i < npair - 1)
        def _():
            prep(c0 + 2, 0)

        finish(c0 + 1, 1)
        return carry

    lax.fori_loop(0, npair, pair_body, 0)

    pltpu.make_async_copy(
        outv.at[0], out_hbm.at[pl.ds(wbase + (nch - 2) * CH, CH)],
        osems[0]).wait()
    pltpu.make_async_copy(
        outv.at[1], out_hbm.at[pl.ds(wbase + (nch - 1) * CH, CH)],
        osems[1]).wait()


def kernel(x, layer1, layer2, layer3, layer4):
    B, H, W, _ = x.shape
    N = B * H * W
    per_w = N // NW
    uv = x.reshape(N, 2).T
    texs = []
    for t in (layer1, layer2, layer3, layer4):
        b = lax.bitcast_convert_type(
            t.reshape(-1).astype(jnp.bfloat16), jnp.uint16).astype(jnp.uint32)
        pk = b | (jnp.roll(b, -1) << 16)
        texs.append(lax.bitcast_convert_type(pk, jnp.int32))
    mesh = plsc.VectorSubcoreMesh(core_axis_name="c", subcore_axis_name="s")
    run = pl.kernel(
        _tex_body,
        out_type=jax.ShapeDtypeStruct((N,), jnp.float32),
        mesh=mesh,
        compiler_params=pltpu.CompilerParams(needs_layout_passes=False),
        scratch_types=[
            pltpu.VMEM((per_w,), jnp.float32),
            pltpu.VMEM((per_w,), jnp.float32),
            pltpu.VMEM((2, 8, CH), jnp.int32),
            pltpu.VMEM((2, NT, CH), jnp.float32),
            pltpu.VMEM((2, 8, CH), jnp.int32),
            pltpu.VMEM((2, CH), jnp.float32),
            pltpu.SemaphoreType.DMA,
            pltpu.SemaphoreType.DMA,
            pltpu.SemaphoreType.DMA,
            pltpu.SemaphoreType.DMA,
        ],
    )
    y = run(uv[0], uv[1], *texs)
    return y.reshape(B, 1, H, W)
